# baseline (speedup 1.0000x reference)
import functools

import jax
import jax.numpy as jnp
from jax import lax
from jax.experimental import pallas as pl
from jax.experimental.pallas import tpu as pltpu
from jax.experimental.pallas import tpu_sc as plsc

N = 10000
E = 320000
IN_F = 128
HID = 128
NCLS = 64

NC = 2
NS = 16
NW = NC * NS
K = 128
ITERS = 80
EPW = K * ITERS
E_PAD = NW * EPW
N_PAD = 10240
NPT = N_PAD // NS

_mesh = plsc.VectorSubcoreMesh(core_axis_name="c", subcore_axis_name="s")


@functools.partial(
    pl.kernel,
    out_type=jax.ShapeDtypeStruct((2 * 2 * N_PAD,), jnp.float32),
    mesh=_mesh,
    scratch_types=[
        pltpu.VMEM((2, K), jnp.int32),
        pltpu.VMEM((2, K), jnp.int32),
        pltpu.VMEM((K,), jnp.float32),
        pltpu.VMEM_SHARED((N_PAD,), jnp.float32),
        pltpu.VMEM_SHARED((N_PAD,), jnp.float32),
        pltpu.SemaphoreType.DMA,
        pltpu.SemaphoreType.DMA,
    ],
)
def _sc_degrees(ei_hbm, zeros_hbm, out_hbm,
                idx_a, idx_b, ones_v, dego_sh, degi_sh, sem_a, sem_b):
    cid = lax.axis_index("c")
    sid = lax.axis_index("s")
    pltpu.sync_copy(zeros_hbm.at[pl.ds(sid * NPT, NPT)],
                    dego_sh.at[pl.ds(sid * NPT, NPT)])
    pltpu.sync_copy(zeros_hbm.at[pl.ds(sid * NPT, NPT)],
                    degi_sh.at[pl.ds(sid * NPT, NPT)])
    for j in range(K // 16):
        ones_v[pl.ds(j * 16, 16)] = jnp.ones((16,), jnp.float32)
    plsc.subcore_barrier()

    base = (cid * NS + sid) * EPW

    def start_idx(buf, sem, t):
        off = base + lax.rem(t, ITERS) * K
        pltpu.async_copy(ei_hbm.at[:, pl.ds(off, K)], buf, sem)

    def wait_idx(buf, sem):
        pltpu.make_async_copy(ei_hbm.at[:, pl.ds(base, K)], buf, sem).wait()

    start_idx(idx_a, sem_a, 0)
    start_idx(idx_b, sem_b, 1)

    def body(j, carry):
        t = 2 * j
        wait_idx(idx_a, sem_a)
        pltpu.sync_copy(ones_v, dego_sh.at[idx_a.at[0]], add=True)
        pltpu.sync_copy(ones_v, degi_sh.at[idx_a.at[1]], add=True)
        start_idx(idx_a, sem_a, t + 2)
        wait_idx(idx_b, sem_b)
        pltpu.sync_copy(ones_v, dego_sh.at[idx_b.at[0]], add=True)
        pltpu.sync_copy(ones_v, degi_sh.at[idx_b.at[1]], add=True)
        start_idx(idx_b, sem_b, t + 3)
        return carry

    lax.fori_loop(0, ITERS // 2, body, 0)
    wait_idx(idx_a, sem_a)
    wait_idx(idx_b, sem_b)

    plsc.subcore_barrier()
    pltpu.sync_copy(dego_sh.at[pl.ds(sid * NPT, NPT)],
                    out_hbm.at[pl.ds(cid * 2 * N_PAD + sid * NPT, NPT)])
    pltpu.sync_copy(degi_sh.at[pl.ds(sid * NPT, NPT)],
                    out_hbm.at[pl.ds(cid * 2 * N_PAD + N_PAD + sid * NPT, NPT)])


@functools.partial(
    pl.kernel,
    out_type=jax.ShapeDtypeStruct((2 * N_PAD, HID), jnp.float32),
    mesh=_mesh,
    scratch_types=[
        pltpu.VMEM((2, K), jnp.int32),
        pltpu.VMEM((2, K), jnp.int32),
        pltpu.VMEM((K, HID), jnp.float32),
        pltpu.VMEM((K, HID), jnp.float32),
        pltpu.VMEM_SHARED((N_PAD, HID), jnp.float32),
        pltpu.SemaphoreType.DMA,
        pltpu.SemaphoreType.DMA,
        pltpu.SemaphoreType.DMA,
        pltpu.SemaphoreType.DMA,
    ],
)
def _sc_segsum(h_hbm, ei_hbm, zeros_hbm, out_hbm,
               idx_a, idx_b, rows_a, rows_b, acc_sh,
               sem_ia, sem_ib, sem_ga, sem_gb):
    cid = lax.axis_index("c")
    sid = lax.axis_index("s")
    pltpu.sync_copy(zeros_hbm.at[pl.ds(sid * NPT, NPT)],
                    acc_sh.at[pl.ds(sid * NPT, NPT)])
    plsc.subcore_barrier()

    base = (cid * NS + sid) * EPW

    def start_idx(buf, sem, t):
        off = base + lax.rem(t, ITERS) * K
        pltpu.async_copy(ei_hbm.at[:, pl.ds(off, K)], buf, sem)

    def wait_idx(buf, sem):
        pltpu.make_async_copy(ei_hbm.at[:, pl.ds(base, K)], buf, sem).wait()

    def start_gather(idx, rows, sem):
        pltpu.async_copy(h_hbm.at[idx.at[0]], rows, sem)

    def wait_gather(idx, rows, sem):
        pltpu.make_async_copy(h_hbm.at[idx.at[0]], rows, sem).wait()

    start_idx(idx_a, sem_ia, 0)
    start_idx(idx_b, sem_ib, 1)
    wait_idx(idx_a, sem_ia)
    start_gather(idx_a, rows_a, sem_ga)

    def body(j, carry):
        t = 2 * j
        wait_idx(idx_b, sem_ib)
        wait_gather(idx_a, rows_a, sem_ga)
        start_gather(idx_b, rows_b, sem_gb)
        pltpu.sync_copy(rows_a, acc_sh.at[idx_a.at[1]], add=True)
        start_idx(idx_a, sem_ia, t + 2)
        wait_idx(idx_a, sem_ia)
        wait_gather(idx_b, rows_b, sem_gb)
        start_gather(idx_a, rows_a, sem_ga)
        pltpu.sync_copy(rows_b, acc_sh.at[idx_b.at[1]], add=True)
        start_idx(idx_b, sem_ib, t + 3)
        return carry

    lax.fori_loop(0, ITERS // 2, body, 0)
    wait_idx(idx_b, sem_ib)
    wait_gather(idx_a, rows_a, sem_ga)

    plsc.subcore_barrier()
    pltpu.sync_copy(acc_sh.at[pl.ds(sid * NPT, NPT)],
                    out_hbm.at[pl.ds(cid * N_PAD + sid * NPT, NPT)])


_BM = 1024
_GRID = N_PAD // _BM


def _mm_body(x_ref, w_ref, o_ref):
    o_ref[...] = jnp.dot(x_ref[...], w_ref[...],
                         preferred_element_type=jnp.float32)


def _tc_matmul(x, w):
    d_in, d_out = w.shape
    return pl.pallas_call(
        _mm_body,
        grid=(_GRID,),
        in_specs=[
            pl.BlockSpec((_BM, d_in), lambda i: (i, 0)),
            pl.BlockSpec((d_in, d_out), lambda i: (0, 0)),
        ],
        out_specs=pl.BlockSpec((_BM, d_out), lambda i: (i, 0)),
        out_shape=jax.ShapeDtypeStruct((N_PAD, d_out), jnp.float32),
    )(x, w)


def _scale_body(z_ref, deg_ref, h_ref, no_ref, ni_ref):
    d = deg_ref[...]
    do = d[0, 0] + d[1, 0]
    di = d[0, 1] + d[1, 1]
    no = lax.rsqrt(jnp.maximum(do, 1.0))
    ni = lax.rsqrt(jnp.maximum(di, 1.0))
    no_ref[...] = no
    ni_ref[...] = ni
    h_ref[...] = z_ref[...] * no


def _tc_scale(z1, degs):
    return pl.pallas_call(
        _scale_body,
        grid=(_GRID,),
        in_specs=[
            pl.BlockSpec((_BM, HID), lambda i: (i, 0)),
            pl.BlockSpec((2, 2, _BM, 1), lambda i: (0, 0, i, 0)),
        ],
        out_specs=[
            pl.BlockSpec((_BM, HID), lambda i: (i, 0)),
            pl.BlockSpec((_BM, 1), lambda i: (i, 0)),
            pl.BlockSpec((_BM, 1), lambda i: (i, 0)),
        ],
        out_shape=[
            jax.ShapeDtypeStruct((N_PAD, HID), jnp.float32),
            jax.ShapeDtypeStruct((N_PAD, 1), jnp.float32),
            jax.ShapeDtypeStruct((N_PAD, 1), jnp.float32),
        ],
    )(z1, degs)


def _layer2_body(p0_ref, p1_ref, ni_ref, no_ref, b1_ref, w2_ref,
                 x1_ref, h2_ref):
    x1 = (p0_ref[...] + p1_ref[...]) * ni_ref[...] + b1_ref[...]
    x1_ref[...] = x1
    x = jnp.maximum(x1, 0.0)
    h2 = jnp.dot(x, w2_ref[...],
                 preferred_element_type=jnp.float32) * no_ref[...]
    h2_ref[...] = jnp.concatenate(
        [h2, jnp.zeros((h2.shape[0], HID - NCLS), jnp.float32)], axis=1)


def _tc_layer2(p0, p1, ni, no, b1, w2):
    return pl.pallas_call(
        _layer2_body,
        grid=(_GRID,),
        in_specs=[
            pl.BlockSpec((_BM, HID), lambda i: (i, 0)),
            pl.BlockSpec((_BM, HID), lambda i: (i, 0)),
            pl.BlockSpec((_BM, 1), lambda i: (i, 0)),
            pl.BlockSpec((_BM, 1), lambda i: (i, 0)),
            pl.BlockSpec((1, HID), lambda i: (0, 0)),
            pl.BlockSpec((HID, NCLS), lambda i: (0, 0)),
        ],
        out_specs=[
            pl.BlockSpec((_BM, HID), lambda i: (i, 0)),
            pl.BlockSpec((_BM, HID), lambda i: (i, 0)),
        ],
        out_shape=[
            jax.ShapeDtypeStruct((N_PAD, HID), jnp.float32),
            jax.ShapeDtypeStruct((N_PAD, HID), jnp.float32),
        ],
    )(p0, p1, ni, no, b1, w2)


def _final_body(q0_ref, q1_ref, ni_ref, b2_ref, o_ref):
    q = q0_ref[...] + q1_ref[...]
    o_ref[...] = q[:, :NCLS] * ni_ref[...] + b2_ref[...]


def _tc_final(q0, q1, ni, b2):
    return pl.pallas_call(
        _final_body,
        grid=(_GRID,),
        in_specs=[
            pl.BlockSpec((_BM, HID), lambda i: (i, 0)),
            pl.BlockSpec((_BM, HID), lambda i: (i, 0)),
            pl.BlockSpec((_BM, 1), lambda i: (i, 0)),
            pl.BlockSpec((1, NCLS), lambda i: (0, 0)),
        ],
        out_specs=pl.BlockSpec((_BM, NCLS), lambda i: (i, 0)),
        out_shape=jax.ShapeDtypeStruct((N_PAD, NCLS), jnp.float32),
    )(q0, q1, ni, b2)


def kernel(features, edge_index, W1, b1, W2, b2):
    ei_pad = jnp.concatenate(
        [edge_index,
         jnp.full((2, E_PAD - E), N_PAD - 1, jnp.int32)], axis=1)
    x_pad = jnp.concatenate(
        [features, jnp.zeros((N_PAD - N, IN_F), jnp.float32)], axis=0)

    zeros_1d = jnp.zeros((N_PAD,), jnp.float32)
    zeros_hid = jnp.zeros((N_PAD, HID), jnp.float32)

    deg_flat = _sc_degrees(ei_pad, zeros_1d)
    degs = deg_flat.reshape(2, 2, N_PAD, 1)

    z1 = _tc_matmul(x_pad, W1)
    h1, no, ni = _tc_scale(z1, degs)

    p = _sc_segsum(h1, ei_pad, zeros_hid).reshape(2, N_PAD, HID)
    x1, h2 = _tc_layer2(p[0], p[1], ni, no, b1.reshape(1, HID), W2)

    q = _sc_segsum(h2, ei_pad, zeros_hid).reshape(2, N_PAD, HID)
    x2 = _tc_final(q[0], q[1], ni, b2.reshape(1, NCLS))

    return (x2[:N], x1[:N])


# speedup vs baseline: 2.4636x; 2.4636x over previous
import functools

import jax
import jax.numpy as jnp
from jax import lax
from jax.experimental import pallas as pl
from jax.experimental.pallas import tpu as pltpu
from jax.experimental.pallas import tpu_sc as plsc

N = 10000
E = 320000
IN_F = 128
HID = 128
NCLS = 64

NC = 2
NS = 16
NW = NC * NS
K = 128
ITERS = 80
EPW = K * ITERS
E_PAD = NW * EPW
N_PAD = 10240
NPT = N_PAD // NS

_mesh = plsc.VectorSubcoreMesh(core_axis_name="c", subcore_axis_name="s")


@functools.partial(
    pl.kernel,
    out_type=jax.ShapeDtypeStruct((2 * 2 * N_PAD,), jnp.float32),
    mesh=_mesh,
    scratch_types=[
        pltpu.VMEM((2, K), jnp.int32),
        pltpu.VMEM((2, K), jnp.int32),
        pltpu.VMEM((K,), jnp.float32),
        pltpu.VMEM_SHARED((N_PAD,), jnp.float32),
        pltpu.VMEM_SHARED((N_PAD,), jnp.float32),
        pltpu.SemaphoreType.DMA,
        pltpu.SemaphoreType.DMA,
    ],
)
def _sc_degrees(ei_hbm, zeros_hbm, out_hbm,
                idx_a, idx_b, ones_v, dego_sh, degi_sh, sem_a, sem_b):
    cid = lax.axis_index("c")
    sid = lax.axis_index("s")
    pltpu.sync_copy(zeros_hbm.at[pl.ds(sid * NPT, NPT)],
                    dego_sh.at[pl.ds(sid * NPT, NPT)])
    pltpu.sync_copy(zeros_hbm.at[pl.ds(sid * NPT, NPT)],
                    degi_sh.at[pl.ds(sid * NPT, NPT)])
    for j in range(K // 16):
        ones_v[pl.ds(j * 16, 16)] = jnp.ones((16,), jnp.float32)
    plsc.subcore_barrier()

    base = (cid * NS + sid) * EPW

    def start_idx(buf, sem, t):
        off = base + lax.rem(t, ITERS) * K
        pltpu.async_copy(ei_hbm.at[:, pl.ds(off, K)], buf, sem)

    def wait_idx(buf, sem):
        pltpu.make_async_copy(ei_hbm.at[:, pl.ds(base, K)], buf, sem).wait()

    start_idx(idx_a, sem_a, 0)
    start_idx(idx_b, sem_b, 1)

    def body(j, carry):
        t = 2 * j
        wait_idx(idx_a, sem_a)
        pltpu.sync_copy(ones_v, dego_sh.at[idx_a.at[0]], add=True)
        pltpu.sync_copy(ones_v, degi_sh.at[idx_a.at[1]], add=True)
        start_idx(idx_a, sem_a, t + 2)
        wait_idx(idx_b, sem_b)
        pltpu.sync_copy(ones_v, dego_sh.at[idx_b.at[0]], add=True)
        pltpu.sync_copy(ones_v, degi_sh.at[idx_b.at[1]], add=True)
        start_idx(idx_b, sem_b, t + 3)
        return carry

    lax.fori_loop(0, ITERS // 2, body, 0)
    wait_idx(idx_a, sem_a)
    wait_idx(idx_b, sem_b)

    plsc.subcore_barrier()
    pltpu.sync_copy(dego_sh.at[pl.ds(sid * NPT, NPT)],
                    out_hbm.at[pl.ds(cid * 2 * N_PAD + sid * NPT, NPT)])
    pltpu.sync_copy(degi_sh.at[pl.ds(sid * NPT, NPT)],
                    out_hbm.at[pl.ds(cid * 2 * N_PAD + N_PAD + sid * NPT, NPT)])


@functools.partial(
    pl.kernel,
    out_type=jax.ShapeDtypeStruct((2 * N_PAD, HID), jnp.float32),
    mesh=_mesh,
    scratch_types=[
        pltpu.VMEM((2, K), jnp.int32),
        pltpu.VMEM((2, K), jnp.int32),
        pltpu.VMEM((K, HID), jnp.float32),
        pltpu.VMEM((K, HID), jnp.float32),
        pltpu.VMEM_SHARED((N_PAD, HID), jnp.float32),
        pltpu.SemaphoreType.DMA,
        pltpu.SemaphoreType.DMA,
        pltpu.SemaphoreType.DMA,
        pltpu.SemaphoreType.DMA,
    ],
)
def _sc_segsum(h_hbm, ei_hbm, zeros_hbm, out_hbm,
               idx_a, idx_b, rows_a, rows_b, acc_sh,
               sem_ia, sem_ib, sem_ga, sem_gb):
    cid = lax.axis_index("c")
    sid = lax.axis_index("s")
    pltpu.sync_copy(zeros_hbm.at[pl.ds(sid * NPT, NPT)],
                    acc_sh.at[pl.ds(sid * NPT, NPT)])
    plsc.subcore_barrier()

    base = (cid * NS + sid) * EPW

    def start_idx(buf, sem, t):
        off = base + lax.rem(t, ITERS) * K
        pltpu.async_copy(ei_hbm.at[:, pl.ds(off, K)], buf, sem)

    def wait_idx(buf, sem):
        pltpu.make_async_copy(ei_hbm.at[:, pl.ds(base, K)], buf, sem).wait()

    def start_gather(idx, rows, sem):
        pltpu.async_copy(h_hbm.at[idx.at[0]], rows, sem)

    def wait_gather(idx, rows, sem):
        pltpu.make_async_copy(h_hbm.at[idx.at[0]], rows, sem).wait()

    start_idx(idx_a, sem_ia, 0)
    start_idx(idx_b, sem_ib, 1)
    wait_idx(idx_a, sem_ia)
    start_gather(idx_a, rows_a, sem_ga)

    def body(j, carry):
        t = 2 * j
        wait_idx(idx_b, sem_ib)
        wait_gather(idx_a, rows_a, sem_ga)
        start_gather(idx_b, rows_b, sem_gb)
        pltpu.sync_copy(rows_a, acc_sh.at[idx_a.at[1]], add=True)
        start_idx(idx_a, sem_ia, t + 2)
        wait_idx(idx_a, sem_ia)
        wait_gather(idx_b, rows_b, sem_gb)
        start_gather(idx_a, rows_a, sem_ga)
        pltpu.sync_copy(rows_b, acc_sh.at[idx_b.at[1]], add=True)
        start_idx(idx_b, sem_ib, t + 3)
        return carry

    lax.fori_loop(0, ITERS // 2, body, 0)
    wait_idx(idx_b, sem_ib)
    wait_gather(idx_a, rows_a, sem_ga)

    plsc.subcore_barrier()
    pltpu.sync_copy(acc_sh.at[pl.ds(sid * NPT, NPT)],
                    out_hbm.at[pl.ds(cid * N_PAD + sid * NPT, NPT)])


_BM = 1024
_GRID = N_PAD // _BM


def _mm_body(x_ref, w_ref, o_ref):
    o_ref[...] = jnp.dot(x_ref[...], w_ref[...],
                         preferred_element_type=jnp.float32)


def _tc_matmul(x, w):
    d_in, d_out = w.shape
    return pl.pallas_call(
        _mm_body,
        grid=(_GRID,),
        in_specs=[
            pl.BlockSpec((_BM, d_in), lambda i: (i, 0)),
            pl.BlockSpec((d_in, d_out), lambda i: (0, 0)),
        ],
        out_specs=pl.BlockSpec((_BM, d_out), lambda i: (i, 0)),
        out_shape=jax.ShapeDtypeStruct((N_PAD, d_out), jnp.float32),
    )(x, w)


def _scale_body(z_ref, deg_ref, h_ref, no_ref, ni_ref):
    d = deg_ref[...]
    do = d[0, 0] + d[1, 0]
    di = d[0, 1] + d[1, 1]
    no = lax.rsqrt(jnp.maximum(do, 1.0))
    ni = lax.rsqrt(jnp.maximum(di, 1.0))
    no_ref[...] = no
    ni_ref[...] = ni
    h_ref[...] = z_ref[...] * no


def _tc_scale(z1, degs):
    return pl.pallas_call(
        _scale_body,
        grid=(_GRID,),
        in_specs=[
            pl.BlockSpec((_BM, HID), lambda i: (i, 0)),
            pl.BlockSpec((2, 2, _BM, 1), lambda i: (0, 0, i, 0)),
        ],
        out_specs=[
            pl.BlockSpec((_BM, HID), lambda i: (i, 0)),
            pl.BlockSpec((_BM, 1), lambda i: (i, 0)),
            pl.BlockSpec((_BM, 1), lambda i: (i, 0)),
        ],
        out_shape=[
            jax.ShapeDtypeStruct((N_PAD, HID), jnp.float32),
            jax.ShapeDtypeStruct((N_PAD, 1), jnp.float32),
            jax.ShapeDtypeStruct((N_PAD, 1), jnp.float32),
        ],
    )(z1, degs)


def _layer2_body(p0_ref, p1_ref, ni_ref, no_ref, b1_ref, w2_ref,
                 x1_ref, h2_ref):
    x1 = (p0_ref[...] + p1_ref[...]) * ni_ref[...] + b1_ref[...]
    x1_ref[...] = x1
    x = jnp.maximum(x1, 0.0)
    h2 = jnp.dot(x, w2_ref[...],
                 preferred_element_type=jnp.float32) * no_ref[...]
    h2_ref[...] = jnp.concatenate(
        [h2, jnp.zeros((h2.shape[0], HID - NCLS), jnp.float32)], axis=1)


def _tc_layer2(p0, p1, ni, no, b1, w2):
    return pl.pallas_call(
        _layer2_body,
        grid=(_GRID,),
        in_specs=[
            pl.BlockSpec((_BM, HID), lambda i: (i, 0)),
            pl.BlockSpec((_BM, HID), lambda i: (i, 0)),
            pl.BlockSpec((_BM, 1), lambda i: (i, 0)),
            pl.BlockSpec((_BM, 1), lambda i: (i, 0)),
            pl.BlockSpec((1, HID), lambda i: (0, 0)),
            pl.BlockSpec((HID, NCLS), lambda i: (0, 0)),
        ],
        out_specs=[
            pl.BlockSpec((_BM, HID), lambda i: (i, 0)),
            pl.BlockSpec((_BM, HID), lambda i: (i, 0)),
        ],
        out_shape=[
            jax.ShapeDtypeStruct((N_PAD, HID), jnp.float32),
            jax.ShapeDtypeStruct((N_PAD, HID), jnp.float32),
        ],
    )(p0, p1, ni, no, b1, w2)


def _final_body(q0_ref, q1_ref, ni_ref, b2_ref, o_ref):
    q = q0_ref[...] + q1_ref[...]
    o_ref[...] = q[:, :NCLS] * ni_ref[...] + b2_ref[...]


def _tc_final(q0, q1, ni, b2):
    return pl.pallas_call(
        _final_body,
        grid=(_GRID,),
        in_specs=[
            pl.BlockSpec((_BM, HID), lambda i: (i, 0)),
            pl.BlockSpec((_BM, HID), lambda i: (i, 0)),
            pl.BlockSpec((_BM, 1), lambda i: (i, 0)),
            pl.BlockSpec((1, NCLS), lambda i: (0, 0)),
        ],
        out_specs=pl.BlockSpec((_BM, NCLS), lambda i: (i, 0)),
        out_shape=jax.ShapeDtypeStruct((N_PAD, NCLS), jnp.float32),
    )(q0, q1, ni, b2)


def kernel(features, edge_index, W1, b1, W2, b2):
    pad_ids = N + jnp.arange(E_PAD - E, dtype=jnp.int32) % (N_PAD - N)
    ei_pad = jnp.concatenate(
        [edge_index, jnp.stack([pad_ids, pad_ids])], axis=1)
    x_pad = jnp.concatenate(
        [features, jnp.zeros((N_PAD - N, IN_F), jnp.float32)], axis=0)

    zeros_1d = jnp.zeros((N_PAD,), jnp.float32)
    zeros_hid = jnp.zeros((N_PAD, HID), jnp.float32)

    deg_flat = _sc_degrees(ei_pad, zeros_1d)
    degs = deg_flat.reshape(2, 2, N_PAD, 1)

    z1 = _tc_matmul(x_pad, W1)
    h1, no, ni = _tc_scale(z1, degs)

    p = _sc_segsum(h1, ei_pad, zeros_hid).reshape(2, N_PAD, HID)
    x1, h2 = _tc_layer2(p[0], p[1], ni, no, b1.reshape(1, HID), W2)

    q = _sc_segsum(h2, ei_pad, zeros_hid).reshape(2, N_PAD, HID)
    x2 = _tc_final(q[0], q[1], ni, b2.reshape(1, NCLS))

    return (x2[:N], x1[:N])


# speedup vs baseline: 2.5851x; 1.0493x over previous
import functools

import jax
import jax.numpy as jnp
from jax import lax
from jax.experimental import pallas as pl
from jax.experimental.pallas import tpu as pltpu
from jax.experimental.pallas import tpu_sc as plsc

N = 10000
E = 320000
IN_F = 128
HID = 128
NCLS = 64

NC = 2
NS = 16
NW = NC * NS
K = 128
ITERS = 80
EPW = K * ITERS
E_PAD = NW * EPW
N_PAD = 10240
NPT = N_PAD // NS

_mesh = plsc.VectorSubcoreMesh(core_axis_name="c", subcore_axis_name="s")


@functools.partial(
    pl.kernel,
    out_type=jax.ShapeDtypeStruct((2 * 2 * N_PAD,), jnp.float32),
    mesh=_mesh,
    scratch_types=[
        pltpu.VMEM((2, K), jnp.int32),
        pltpu.VMEM((2, K), jnp.int32),
        pltpu.VMEM((K,), jnp.float32),
        pltpu.VMEM_SHARED((N_PAD,), jnp.float32),
        pltpu.VMEM_SHARED((N_PAD,), jnp.float32),
        pltpu.SemaphoreType.DMA,
        pltpu.SemaphoreType.DMA,
    ],
)
def _sc_degrees(ei_hbm, zeros_hbm, out_hbm,
                idx_a, idx_b, ones_v, dego_sh, degi_sh, sem_a, sem_b):
    cid = lax.axis_index("c")
    sid = lax.axis_index("s")
    pltpu.sync_copy(zeros_hbm.at[pl.ds(sid * NPT, NPT)],
                    dego_sh.at[pl.ds(sid * NPT, NPT)])
    pltpu.sync_copy(zeros_hbm.at[pl.ds(sid * NPT, NPT)],
                    degi_sh.at[pl.ds(sid * NPT, NPT)])
    for j in range(K // 16):
        ones_v[pl.ds(j * 16, 16)] = jnp.ones((16,), jnp.float32)
    plsc.subcore_barrier()

    base = (cid * NS + sid) * EPW

    def start_idx(buf, sem, t):
        off = base + lax.rem(t, ITERS) * K
        pltpu.async_copy(ei_hbm.at[:, pl.ds(off, K)], buf, sem)

    def wait_idx(buf, sem):
        pltpu.make_async_copy(ei_hbm.at[:, pl.ds(base, K)], buf, sem).wait()

    start_idx(idx_a, sem_a, 0)
    start_idx(idx_b, sem_b, 1)

    def body(j, carry):
        t = 2 * j
        wait_idx(idx_a, sem_a)
        pltpu.sync_copy(ones_v, dego_sh.at[idx_a.at[0]], add=True)
        pltpu.sync_copy(ones_v, degi_sh.at[idx_a.at[1]], add=True)
        start_idx(idx_a, sem_a, t + 2)
        wait_idx(idx_b, sem_b)
        pltpu.sync_copy(ones_v, dego_sh.at[idx_b.at[0]], add=True)
        pltpu.sync_copy(ones_v, degi_sh.at[idx_b.at[1]], add=True)
        start_idx(idx_b, sem_b, t + 3)
        return carry

    lax.fori_loop(0, ITERS // 2, body, 0)
    wait_idx(idx_a, sem_a)
    wait_idx(idx_b, sem_b)

    plsc.subcore_barrier()
    pltpu.sync_copy(dego_sh.at[pl.ds(sid * NPT, NPT)],
                    out_hbm.at[pl.ds(cid * 2 * N_PAD + sid * NPT, NPT)])
    pltpu.sync_copy(degi_sh.at[pl.ds(sid * NPT, NPT)],
                    out_hbm.at[pl.ds(cid * 2 * N_PAD + N_PAD + sid * NPT, NPT)])


def _make_segsum(D, tc_tiling):
    @functools.partial(
        pl.kernel,
        out_type=jax.ShapeDtypeStruct((2 * N_PAD, D), jnp.float32),
        mesh=_mesh,
        compiler_params=pltpu.CompilerParams(use_tc_tiling_on_sc=tc_tiling),
        scratch_types=[
            pltpu.VMEM((2, K), jnp.int32),
            pltpu.VMEM((2, K), jnp.int32),
            pltpu.VMEM((K, D), jnp.float32),
            pltpu.VMEM((K, D), jnp.float32),
            pltpu.VMEM_SHARED((N_PAD, D), jnp.float32),
            pltpu.SemaphoreType.DMA,
            pltpu.SemaphoreType.DMA,
            pltpu.SemaphoreType.DMA,
            pltpu.SemaphoreType.DMA,
        ],
    )
    def segsum(h_hbm, ei_hbm, zeros_hbm, out_hbm,
               idx_a, idx_b, rows_a, rows_b, acc_sh,
               sem_ia, sem_ib, sem_ga, sem_gb):
        cid = lax.axis_index("c")
        sid = lax.axis_index("s")
        pltpu.sync_copy(zeros_hbm.at[pl.ds(sid * NPT, NPT)],
                        acc_sh.at[pl.ds(sid * NPT, NPT)])
        plsc.subcore_barrier()

        base = (cid * NS + sid) * EPW

        def start_idx(buf, sem, t):
            off = base + lax.rem(t, ITERS) * K
            pltpu.async_copy(ei_hbm.at[:, pl.ds(off, K)], buf, sem)

        def wait_idx(buf, sem):
            pltpu.make_async_copy(ei_hbm.at[:, pl.ds(base, K)], buf, sem).wait()

        def start_gather(idx, rows, sem):
            pltpu.async_copy(h_hbm.at[idx.at[0]], rows, sem)

        def wait_gather(idx, rows, sem):
            pltpu.make_async_copy(h_hbm.at[idx.at[0]], rows, sem).wait()

        start_idx(idx_a, sem_ia, 0)
        start_idx(idx_b, sem_ib, 1)
        wait_idx(idx_a, sem_ia)
        start_gather(idx_a, rows_a, sem_ga)

        def body(j, carry):
            t = 2 * j
            wait_idx(idx_b, sem_ib)
            wait_gather(idx_a, rows_a, sem_ga)
            start_gather(idx_b, rows_b, sem_gb)
            pltpu.sync_copy(rows_a, acc_sh.at[idx_a.at[1]], add=True)
            start_idx(idx_a, sem_ia, t + 2)
            wait_idx(idx_a, sem_ia)
            wait_gather(idx_b, rows_b, sem_gb)
            start_gather(idx_a, rows_a, sem_ga)
            pltpu.sync_copy(rows_b, acc_sh.at[idx_b.at[1]], add=True)
            start_idx(idx_b, sem_ib, t + 3)
            return carry

        lax.fori_loop(0, ITERS // 2, body, 0)
        wait_idx(idx_b, sem_ib)
        wait_gather(idx_a, rows_a, sem_ga)

        plsc.subcore_barrier()
        pltpu.sync_copy(acc_sh.at[pl.ds(sid * NPT, NPT)],
                        out_hbm.at[pl.ds(cid * N_PAD + sid * NPT, NPT)])

    return segsum


_sc_segsum_hid = _make_segsum(HID, True)
_sc_segsum_cls = _make_segsum(NCLS, False)


_BM = 1024
_GRID = N_PAD // _BM


def _mm_body(x_ref, w_ref, o_ref):
    o_ref[...] = jnp.dot(x_ref[...], w_ref[...],
                         preferred_element_type=jnp.float32)


def _tc_matmul(x, w):
    d_in, d_out = w.shape
    return pl.pallas_call(
        _mm_body,
        grid=(_GRID,),
        in_specs=[
            pl.BlockSpec((_BM, d_in), lambda i: (i, 0)),
            pl.BlockSpec((d_in, d_out), lambda i: (0, 0)),
        ],
        out_specs=pl.BlockSpec((_BM, d_out), lambda i: (i, 0)),
        out_shape=jax.ShapeDtypeStruct((N_PAD, d_out), jnp.float32),
    )(x, w)


def _scale_body(z_ref, deg_ref, h_ref, no_ref, ni_ref):
    d = deg_ref[...]
    do = d[0, 0] + d[1, 0]
    di = d[0, 1] + d[1, 1]
    no = lax.rsqrt(jnp.maximum(do, 1.0))
    ni = lax.rsqrt(jnp.maximum(di, 1.0))
    no_ref[...] = no
    ni_ref[...] = ni
    h_ref[...] = z_ref[...] * no


def _tc_scale(z1, degs):
    return pl.pallas_call(
        _scale_body,
        grid=(_GRID,),
        in_specs=[
            pl.BlockSpec((_BM, HID), lambda i: (i, 0)),
            pl.BlockSpec((2, 2, _BM, 1), lambda i: (0, 0, i, 0)),
        ],
        out_specs=[
            pl.BlockSpec((_BM, HID), lambda i: (i, 0)),
            pl.BlockSpec((_BM, 1), lambda i: (i, 0)),
            pl.BlockSpec((_BM, 1), lambda i: (i, 0)),
        ],
        out_shape=[
            jax.ShapeDtypeStruct((N_PAD, HID), jnp.float32),
            jax.ShapeDtypeStruct((N_PAD, 1), jnp.float32),
            jax.ShapeDtypeStruct((N_PAD, 1), jnp.float32),
        ],
    )(z1, degs)


def _layer2_body(p0_ref, p1_ref, ni_ref, no_ref, b1_ref, w2_ref,
                 x1_ref, h2_ref):
    x1 = (p0_ref[...] + p1_ref[...]) * ni_ref[...] + b1_ref[...]
    x1_ref[...] = x1
    x = jnp.maximum(x1, 0.0)
    h2_ref[...] = jnp.dot(x, w2_ref[...],
                          preferred_element_type=jnp.float32) * no_ref[...]


def _tc_layer2(p0, p1, ni, no, b1, w2):
    return pl.pallas_call(
        _layer2_body,
        grid=(_GRID,),
        in_specs=[
            pl.BlockSpec((_BM, HID), lambda i: (i, 0)),
            pl.BlockSpec((_BM, HID), lambda i: (i, 0)),
            pl.BlockSpec((_BM, 1), lambda i: (i, 0)),
            pl.BlockSpec((_BM, 1), lambda i: (i, 0)),
            pl.BlockSpec((1, HID), lambda i: (0, 0)),
            pl.BlockSpec((HID, NCLS), lambda i: (0, 0)),
        ],
        out_specs=[
            pl.BlockSpec((_BM, HID), lambda i: (i, 0)),
            pl.BlockSpec((_BM, NCLS), lambda i: (i, 0)),
        ],
        out_shape=[
            jax.ShapeDtypeStruct((N_PAD, HID), jnp.float32),
            jax.ShapeDtypeStruct((N_PAD, NCLS), jnp.float32),
        ],
    )(p0, p1, ni, no, b1, w2)


def _final_body(q0_ref, q1_ref, ni_ref, b2_ref, o_ref):
    q = q0_ref[...] + q1_ref[...]
    o_ref[...] = q * ni_ref[...] + b2_ref[...]


def _tc_final(q0, q1, ni, b2):
    return pl.pallas_call(
        _final_body,
        grid=(_GRID,),
        in_specs=[
            pl.BlockSpec((_BM, NCLS), lambda i: (i, 0)),
            pl.BlockSpec((_BM, NCLS), lambda i: (i, 0)),
            pl.BlockSpec((_BM, 1), lambda i: (i, 0)),
            pl.BlockSpec((1, NCLS), lambda i: (0, 0)),
        ],
        out_specs=pl.BlockSpec((_BM, NCLS), lambda i: (i, 0)),
        out_shape=jax.ShapeDtypeStruct((N_PAD, NCLS), jnp.float32),
    )(q0, q1, ni, b2)


def kernel(features, edge_index, W1, b1, W2, b2):
    pad_ids = N + jnp.arange(E_PAD - E, dtype=jnp.int32) % (N_PAD - N)
    ei_pad = jnp.concatenate(
        [edge_index, jnp.stack([pad_ids, pad_ids])], axis=1)
    x_pad = jnp.concatenate(
        [features, jnp.zeros((N_PAD - N, IN_F), jnp.float32)], axis=0)

    zeros_1d = jnp.zeros((N_PAD,), jnp.float32)
    zeros_hid = jnp.zeros((N_PAD, HID), jnp.float32)
    zeros_cls = jnp.zeros((N_PAD, NCLS), jnp.float32)

    deg_flat = _sc_degrees(ei_pad, zeros_1d)
    degs = deg_flat.reshape(2, 2, N_PAD, 1)

    z1 = _tc_matmul(x_pad, W1)
    h1, no, ni = _tc_scale(z1, degs)

    p = _sc_segsum_hid(h1, ei_pad, zeros_hid).reshape(2, N_PAD, HID)
    x1, h2 = _tc_layer2(p[0], p[1], ni, no, b1.reshape(1, HID), W2)

    q = _sc_segsum_cls(h2, ei_pad, zeros_cls).reshape(2, N_PAD, NCLS)
    x2 = _tc_final(q[0], q[1], ni, b2.reshape(1, NCLS))

    return (x2[:N], x1[:N])


# speedup vs baseline: 2.9889x; 1.1562x over previous
import functools

import jax
import jax.numpy as jnp
from jax import lax
from jax.experimental import pallas as pl
from jax.experimental.pallas import tpu as pltpu
from jax.experimental.pallas import tpu_sc as plsc

N = 10000
E = 320000
IN_F = 128
HID = 128
NCLS = 64

NC = 2
NS = 16
NW = NC * NS
K = 128
ITERS = 80
EPW = K * ITERS
E_PAD = NW * EPW
N_PAD = 10240
NPT = N_PAD // NS

_mesh = plsc.VectorSubcoreMesh(core_axis_name="c", subcore_axis_name="s")


@functools.partial(
    pl.kernel,
    out_type=jax.ShapeDtypeStruct((2 * 2 * N_PAD,), jnp.float32),
    mesh=_mesh,
    scratch_types=[
        pltpu.VMEM((2, K), jnp.int32),
        pltpu.VMEM((2, K), jnp.int32),
        pltpu.VMEM((K,), jnp.float32),
        pltpu.VMEM_SHARED((N_PAD,), jnp.float32),
        pltpu.VMEM_SHARED((N_PAD,), jnp.float32),
        pltpu.SemaphoreType.DMA,
        pltpu.SemaphoreType.DMA,
    ],
)
def _sc_degrees(ei_hbm, zeros_hbm, out_hbm,
                idx_a, idx_b, ones_v, dego_sh, degi_sh, sem_a, sem_b):
    cid = lax.axis_index("c")
    sid = lax.axis_index("s")
    pltpu.sync_copy(zeros_hbm.at[pl.ds(sid * NPT, NPT)],
                    dego_sh.at[pl.ds(sid * NPT, NPT)])
    pltpu.sync_copy(zeros_hbm.at[pl.ds(sid * NPT, NPT)],
                    degi_sh.at[pl.ds(sid * NPT, NPT)])
    for j in range(K // 16):
        ones_v[pl.ds(j * 16, 16)] = jnp.ones((16,), jnp.float32)
    plsc.subcore_barrier()

    base = (cid * NS + sid) * EPW

    def start_idx(buf, sem, t):
        off = base + lax.rem(t, ITERS) * K
        pltpu.async_copy(ei_hbm.at[:, pl.ds(off, K)], buf, sem)

    def wait_idx(buf, sem):
        pltpu.make_async_copy(ei_hbm.at[:, pl.ds(base, K)], buf, sem).wait()

    start_idx(idx_a, sem_a, 0)
    start_idx(idx_b, sem_b, 1)

    def body(j, carry):
        t = 2 * j
        wait_idx(idx_a, sem_a)
        pltpu.sync_copy(ones_v, dego_sh.at[idx_a.at[0]], add=True)
        pltpu.sync_copy(ones_v, degi_sh.at[idx_a.at[1]], add=True)
        start_idx(idx_a, sem_a, t + 2)
        wait_idx(idx_b, sem_b)
        pltpu.sync_copy(ones_v, dego_sh.at[idx_b.at[0]], add=True)
        pltpu.sync_copy(ones_v, degi_sh.at[idx_b.at[1]], add=True)
        start_idx(idx_b, sem_b, t + 3)
        return carry

    lax.fori_loop(0, ITERS // 2, body, 0)
    wait_idx(idx_a, sem_a)
    wait_idx(idx_b, sem_b)

    plsc.subcore_barrier()
    pltpu.sync_copy(dego_sh.at[pl.ds(sid * NPT, NPT)],
                    out_hbm.at[pl.ds(cid * N_PAD + sid * NPT, NPT)])
    pltpu.sync_copy(degi_sh.at[pl.ds(sid * NPT, NPT)],
                    out_hbm.at[pl.ds(2 * N_PAD + cid * N_PAD + sid * NPT, NPT)])


def _make_segsum(D, tc_tiling):
    @functools.partial(
        pl.kernel,
        out_type=jax.ShapeDtypeStruct((2 * N_PAD, D), jnp.float32),
        mesh=_mesh,
        compiler_params=pltpu.CompilerParams(use_tc_tiling_on_sc=tc_tiling),
        scratch_types=[
            pltpu.VMEM((2, K), jnp.int32),
            pltpu.VMEM((2, K), jnp.int32),
            pltpu.VMEM((K, D), jnp.float32),
            pltpu.VMEM((K, D), jnp.float32),
            pltpu.VMEM_SHARED((N_PAD, D), jnp.float32),
            pltpu.SemaphoreType.DMA,
            pltpu.SemaphoreType.DMA,
            pltpu.SemaphoreType.DMA,
            pltpu.SemaphoreType.DMA,
        ],
    )
    def segsum(h_hbm, ei_hbm, zeros_hbm, out_hbm,
               idx_a, idx_b, rows_a, rows_b, acc_sh,
               sem_ia, sem_ib, sem_ga, sem_gb):
        cid = lax.axis_index("c")
        sid = lax.axis_index("s")
        pltpu.sync_copy(zeros_hbm.at[pl.ds(sid * NPT, NPT)],
                        acc_sh.at[pl.ds(sid * NPT, NPT)])
        plsc.subcore_barrier()

        base = (cid * NS + sid) * EPW

        def start_idx(buf, sem, t):
            off = base + lax.rem(t, ITERS) * K
            pltpu.async_copy(ei_hbm.at[:, pl.ds(off, K)], buf, sem)

        def wait_idx(buf, sem):
            pltpu.make_async_copy(ei_hbm.at[:, pl.ds(base, K)], buf, sem).wait()

        def start_gather(idx, rows, sem):
            pltpu.async_copy(h_hbm.at[idx.at[0]], rows, sem)

        def wait_gather(idx, rows, sem):
            pltpu.make_async_copy(h_hbm.at[idx.at[0]], rows, sem).wait()

        start_idx(idx_a, sem_ia, 0)
        start_idx(idx_b, sem_ib, 1)
        wait_idx(idx_a, sem_ia)
        start_gather(idx_a, rows_a, sem_ga)

        def body(j, carry):
            t = 2 * j
            wait_idx(idx_b, sem_ib)
            wait_gather(idx_a, rows_a, sem_ga)
            start_gather(idx_b, rows_b, sem_gb)
            pltpu.sync_copy(rows_a, acc_sh.at[idx_a.at[1]], add=True)
            start_idx(idx_a, sem_ia, t + 2)
            wait_idx(idx_a, sem_ia)
            wait_gather(idx_b, rows_b, sem_gb)
            start_gather(idx_a, rows_a, sem_ga)
            pltpu.sync_copy(rows_b, acc_sh.at[idx_b.at[1]], add=True)
            start_idx(idx_b, sem_ib, t + 3)
            return carry

        lax.fori_loop(0, ITERS // 2, body, 0)
        wait_idx(idx_b, sem_ib)
        wait_gather(idx_a, rows_a, sem_ga)

        plsc.subcore_barrier()
        pltpu.sync_copy(acc_sh.at[pl.ds(sid * NPT, NPT)],
                        out_hbm.at[pl.ds(cid * N_PAD + sid * NPT, NPT)])

    return segsum


_sc_segsum_hid = _make_segsum(HID, True)
_sc_segsum_cls = _make_segsum(NCLS, False)


_BM = 1024
_GRID = N_PAD // _BM


def _col(v):
    return jnp.transpose(v, (1, 0))


def _mm_body(x_ref, w_ref, o_ref):
    o_ref[...] = jnp.dot(x_ref[...], w_ref[...],
                         preferred_element_type=jnp.float32)


def _tc_matmul(x, w):
    d_in, d_out = w.shape
    return pl.pallas_call(
        _mm_body,
        grid=(_GRID,),
        in_specs=[
            pl.BlockSpec((_BM, d_in), lambda i: (i, 0)),
            pl.BlockSpec((d_in, d_out), lambda i: (0, 0)),
        ],
        out_specs=pl.BlockSpec((_BM, d_out), lambda i: (i, 0)),
        out_shape=jax.ShapeDtypeStruct((N_PAD, d_out), jnp.float32),
    )(x, w)


def _scale_body(z_ref, deg_ref, h_ref):
    d = deg_ref[...]
    no = _col(lax.rsqrt(jnp.maximum(d[0:1] + d[1:2], 1.0)))
    h_ref[...] = z_ref[...] * no


def _tc_scale(z1, degs):
    return pl.pallas_call(
        _scale_body,
        grid=(_GRID,),
        in_specs=[
            pl.BlockSpec((_BM, HID), lambda i: (i, 0)),
            pl.BlockSpec((4, _BM), lambda i: (0, i)),
        ],
        out_specs=pl.BlockSpec((_BM, HID), lambda i: (i, 0)),
        out_shape=jax.ShapeDtypeStruct((N_PAD, HID), jnp.float32),
    )(z1, degs)


def _layer2_body(p0_ref, p1_ref, deg_ref, b1_ref, w2_ref, x1_ref, h2_ref):
    d = deg_ref[...]
    no = _col(lax.rsqrt(jnp.maximum(d[0:1] + d[1:2], 1.0)))
    ni = _col(lax.rsqrt(jnp.maximum(d[2:3] + d[3:4], 1.0)))
    x1 = (p0_ref[...] + p1_ref[...]) * ni + b1_ref[...]
    x1_ref[...] = x1
    x = jnp.maximum(x1, 0.0)
    h2_ref[...] = jnp.dot(x, w2_ref[...],
                          preferred_element_type=jnp.float32) * no


def _tc_layer2(p, degs, b1, w2):
    return pl.pallas_call(
        _layer2_body,
        grid=(_GRID,),
        in_specs=[
            pl.BlockSpec((_BM, HID), lambda i: (i, 0)),
            pl.BlockSpec((_BM, HID), lambda i: (i + _GRID, 0)),
            pl.BlockSpec((4, _BM), lambda i: (0, i)),
            pl.BlockSpec((1, HID), lambda i: (0, 0)),
            pl.BlockSpec((HID, NCLS), lambda i: (0, 0)),
        ],
        out_specs=[
            pl.BlockSpec((_BM, HID), lambda i: (i, 0)),
            pl.BlockSpec((_BM, NCLS), lambda i: (i, 0)),
        ],
        out_shape=[
            jax.ShapeDtypeStruct((N, HID), jnp.float32),
            jax.ShapeDtypeStruct((N_PAD, NCLS), jnp.float32),
        ],
    )(p, p, degs, b1, w2)


def _final_body(q0_ref, q1_ref, deg_ref, b2_ref, o_ref):
    d = deg_ref[...]
    ni = _col(lax.rsqrt(jnp.maximum(d[2:3] + d[3:4], 1.0)))
    o_ref[...] = (q0_ref[...] + q1_ref[...]) * ni + b2_ref[...]


def _tc_final(q, degs, b2):
    return pl.pallas_call(
        _final_body,
        grid=(_GRID,),
        in_specs=[
            pl.BlockSpec((_BM, NCLS), lambda i: (i, 0)),
            pl.BlockSpec((_BM, NCLS), lambda i: (i + _GRID, 0)),
            pl.BlockSpec((4, _BM), lambda i: (0, i)),
            pl.BlockSpec((1, NCLS), lambda i: (0, 0)),
        ],
        out_specs=pl.BlockSpec((_BM, NCLS), lambda i: (i, 0)),
        out_shape=jax.ShapeDtypeStruct((N, NCLS), jnp.float32),
    )(q, q, degs, b2)


def kernel(features, edge_index, W1, b1, W2, b2):
    pad_ids = N + jnp.arange(E_PAD - E, dtype=jnp.int32) % (N_PAD - N)
    ei_pad = jnp.concatenate(
        [edge_index, jnp.stack([pad_ids, pad_ids])], axis=1)
    x_pad = jnp.concatenate(
        [features, jnp.zeros((N_PAD - N, IN_F), jnp.float32)], axis=0)

    zeros_1d = jnp.zeros((N_PAD,), jnp.float32)
    zeros_hid = jnp.zeros((N_PAD, HID), jnp.float32)
    zeros_cls = jnp.zeros((N_PAD, NCLS), jnp.float32)

    degs = _sc_degrees(ei_pad, zeros_1d).reshape(4, N_PAD)

    z1 = _tc_matmul(x_pad, W1)
    h1 = _tc_scale(z1, degs)

    p = _sc_segsum_hid(h1, ei_pad, zeros_hid)
    x1, h2 = _tc_layer2(p, degs, b1.reshape(1, HID), W2)

    q = _sc_segsum_cls(h2, ei_pad, zeros_cls)
    x2 = _tc_final(q, degs, b2.reshape(1, NCLS))

    return (x2, x1)
